# initial kernel scaffold (unmeasured)
import jax
import jax.numpy as jnp
from jax import lax
from jax.experimental import pallas as pl
from jax.experimental.pallas import tpu as pltpu

N_Z = 4


def kernel(Q, K, V):
    b, qlen, h, d = Q.shape
    k_per = K.shape[1]
    scale = d ** -0.5

    def body(q_ref, k_ref, v_ref, out_ref, o_comm, s_comm,
             o_send, o_recv, s_send, s_recv):
        i = pl.program_id(0)
        nb = pl.num_programs(0)

        q = q_ref[i, 0]
        k = k_ref[0]
        v = v_ref[0]
        s_t = jnp.sum(q[None, :, :] * k, axis=-1) * scale
        m = jnp.max(s_t, axis=0)
        p_t = jnp.exp(s_t - m[None, :])
        l = jnp.sum(p_t, axis=0)
        o = jnp.sum(p_t[:, :, None] * v, axis=0)

        o_comm[0, i] = o
        s_comm[0, 0, i] = m
        s_comm[0, 1, i] = l

        @pl.when(i == nb - 1)
        def _comm():
            my_x = lax.axis_index("x")
            my_y = lax.axis_index("y")
            my_z = lax.axis_index("z")

            barrier = pltpu.get_barrier_semaphore()
            for off in (1, 2, 3):
                dst_z = lax.rem(my_z + off, N_Z)
                pl.semaphore_signal(
                    barrier, inc=1, device_id=(my_x, my_y, dst_z),
                    device_id_type=pl.DeviceIdType.MESH)
            pl.semaphore_wait(barrier, N_Z - 1)

            sends = []
            for off in (1, 2, 3):
                dst_z = lax.rem(my_z + off, N_Z)
                slot = N_Z - off
                for comm, ssem, rsem in ((o_comm, o_send, o_recv),
                                         (s_comm, s_send, s_recv)):
                    rd = pltpu.make_async_remote_copy(
                        src_ref=comm.at[0],
                        dst_ref=comm.at[slot],
                        send_sem=ssem.at[off - 1],
                        recv_sem=rsem.at[slot - 1],
                        device_id=(my_x, my_y, dst_z),
                        device_id_type=pl.DeviceIdType.MESH)
                    rd.start()
                    sends.append(rd)

            for slot in (1, 2, 3):
                for comm, ssem, rsem in ((o_comm, o_send, o_recv),
                                         (s_comm, s_send, s_recv)):
                    rd = pltpu.make_async_remote_copy(
                        src_ref=comm.at[0],
                        dst_ref=comm.at[slot],
                        send_sem=ssem.at[0],
                        recv_sem=rsem.at[slot - 1],
                        device_id=(my_x, my_y, my_z),
                        device_id_type=pl.DeviceIdType.MESH)
                    rd.wait_recv()

            for rd in sends:
                rd.wait_send()

            m_all = s_comm[:, 0]
            l_all = s_comm[:, 1]
            mx = jnp.max(m_all, axis=0)
            alpha = jnp.exp(m_all - mx[None])
            l_tot = jnp.sum(l_all * alpha, axis=0)
            o_all = o_comm[:, :, :, :]
            o_fin = (jnp.sum(o_all * alpha[..., None], axis=0)
                     / l_tot[..., None])
            out_ref[:, 0, :, :] = o_fin

    return pl.pallas_call(
        body,
        grid=(b,),
        in_specs=[
            pl.BlockSpec((b, qlen, h, d), lambda i: (0, 0, 0, 0)),
            pl.BlockSpec((1, k_per, h, d), lambda i: (i, 0, 0, 0)),
            pl.BlockSpec((1, k_per, h, d), lambda i: (i, 0, 0, 0)),
        ],
        out_specs=pl.BlockSpec((b, qlen, h, d), lambda i: (0, 0, 0, 0)),
        out_shape=jax.ShapeDtypeStruct((b, qlen, h, d), jnp.float32),
        scratch_shapes=[
            pltpu.VMEM((N_Z, b, h, d), jnp.float32),
            pltpu.VMEM((N_Z, 2, b, h), jnp.float32),
            pltpu.SemaphoreType.DMA((N_Z - 1,)),
            pltpu.SemaphoreType.DMA((N_Z - 1,)),
            pltpu.SemaphoreType.DMA((N_Z - 1,)),
            pltpu.SemaphoreType.DMA((N_Z - 1,)),
        ],
        compiler_params=pltpu.CompilerParams(
            collective_id=0,
            dimension_semantics=("arbitrary",),
        ),
    )(Q, K, V)


# baseline (device time: 344125 ns/iter reference)
import jax
import jax.numpy as jnp
from jax import lax
from jax.experimental import pallas as pl
from jax.experimental.pallas import tpu as pltpu

N_Z = 4


def kernel(Q, K, V):
    b, qlen, h, d = Q.shape
    k_per = K.shape[1]
    scale = d ** -0.5

    def body(q_ref, k_ref, v_ref, out_ref, o_comm, s_comm,
             o_send, o_recv, s_send, s_recv):
        i = pl.program_id(0)
        nb = pl.num_programs(0)

        q = q_ref[i, 0]
        k = k_ref[0]
        v = v_ref[0]
        s_t = jnp.sum(q[None, :, :] * k, axis=-1) * scale
        m = jnp.max(s_t, axis=0)
        p_t = jnp.exp(s_t - m[None, :])
        l = jnp.sum(p_t, axis=0)
        o = jnp.sum(p_t[:, :, None] * v, axis=0)

        o_comm[0, i] = o
        s_comm[0, 0, i] = m
        s_comm[0, 1, i] = l

        @pl.when(i == nb - 1)
        def _comm():
            my_x = lax.axis_index("x")
            my_y = lax.axis_index("y")
            my_z = lax.axis_index("z")

            barrier = pltpu.get_barrier_semaphore()
            for off in (1, 2, 3):
                dst_z = lax.rem(my_z + off, N_Z)
                pl.semaphore_signal(
                    barrier, inc=1, device_id=(my_x, my_y, dst_z),
                    device_id_type=pl.DeviceIdType.MESH)
            pl.semaphore_wait(barrier, N_Z - 1)

            sends = []
            for off in (1, 2, 3):
                dst_z = lax.rem(my_z + off, N_Z)
                slot = N_Z - off
                for comm, ssem, rsem in ((o_comm, o_send, o_recv),
                                         (s_comm, s_send, s_recv)):
                    rd = pltpu.make_async_remote_copy(
                        src_ref=comm.at[0],
                        dst_ref=comm.at[slot],
                        send_sem=ssem.at[off - 1],
                        recv_sem=rsem.at[slot - 1],
                        device_id=(my_x, my_y, dst_z),
                        device_id_type=pl.DeviceIdType.MESH)
                    rd.start()
                    sends.append(rd)

            for slot in (1, 2, 3):
                for comm, ssem, rsem in ((o_comm, o_send, o_recv),
                                         (s_comm, s_send, s_recv)):
                    rd = pltpu.make_async_remote_copy(
                        src_ref=comm.at[0],
                        dst_ref=comm.at[slot],
                        send_sem=ssem.at[0],
                        recv_sem=rsem.at[slot - 1],
                        device_id=(my_x, my_y, my_z),
                        device_id_type=pl.DeviceIdType.MESH)
                    rd.wait_recv()

            for rd in sends:
                rd.wait_send()

            m_all = s_comm[:, 0]
            l_all = s_comm[:, 1]
            mx = jnp.max(m_all, axis=0)
            alpha = jnp.exp(m_all - mx[None])
            l_tot = jnp.sum(l_all * alpha, axis=0)
            o_all = o_comm[:, :, :, :]
            o_fin = (jnp.sum(o_all * alpha[..., None], axis=0)
                     / l_tot[..., None])
            out_ref[:, 0, :, :] = o_fin

    return pl.pallas_call(
        body,
        grid=(b,),
        in_specs=[
            pl.BlockSpec((b, qlen, h, d), lambda i: (0, 0, 0, 0)),
            pl.BlockSpec((1, k_per, h, d), lambda i: (i, 0, 0, 0)),
            pl.BlockSpec((1, k_per, h, d), lambda i: (i, 0, 0, 0)),
        ],
        out_specs=pl.BlockSpec((b, qlen, h, d), lambda i: (0, 0, 0, 0)),
        out_shape=jax.ShapeDtypeStruct((b, qlen, h, d), jnp.float32),
        scratch_shapes=[
            pltpu.VMEM((N_Z, b, h, d), jnp.float32),
            pltpu.VMEM((N_Z, 2, b, h), jnp.float32),
            pltpu.SemaphoreType.DMA((N_Z - 1,)),
            pltpu.SemaphoreType.DMA((N_Z - 1,)),
            pltpu.SemaphoreType.DMA((N_Z - 1,)),
            pltpu.SemaphoreType.DMA((N_Z - 1,)),
        ],
        compiler_params=pltpu.CompilerParams(
            collective_id=0,
            dimension_semantics=("arbitrary",),
            vmem_limit_bytes=64 * 1024 * 1024,
        ),
    )(Q, K, V)


# device time: 318080 ns/iter; 1.0819x vs baseline; 1.0819x over previous
import jax
import jax.numpy as jnp
from jax import lax
from jax.experimental import pallas as pl
from jax.experimental.pallas import tpu as pltpu

N_Z = 4


def kernel(Q, K, V):
    b, qlen, h, d = Q.shape
    k_per = K.shape[1]
    scale = d ** -0.5

    def body(q_ref, k_ref, v_ref, out_ref, o_comm, s_comm,
             o_send, o_recv, s_send, s_recv):
        i = pl.program_id(0)
        nb = pl.num_programs(0)

        hd = h * d
        q = q_ref[i, 0]
        k2 = jnp.reshape(k_ref[0], (k_per, hd))
        v2 = jnp.reshape(v_ref[0], (k_per, hd))

        hh = lax.broadcasted_iota(jnp.int32, (h, d, h), 0)
        hc = lax.broadcasted_iota(jnp.int32, (h, d, h), 2)
        qdiag = jnp.reshape(
            jnp.where(hh == hc, q[:, :, None], 0.0), (hd, h))
        er = lax.broadcasted_iota(jnp.int32, (h, hd), 0)
        ec = lax.broadcasted_iota(jnp.int32, (h, hd), 1) // d
        expander = jnp.where(er == ec, 1.0, 0.0).astype(jnp.float32)

        s = lax.dot_general(
            k2, qdiag, (((1,), (0,)), ((), ())),
            preferred_element_type=jnp.float32) * scale
        m = jnp.max(s, axis=0)
        p = jnp.exp(s - m[None, :])
        l = jnp.sum(p, axis=0)
        p_exp = lax.dot_general(
            p, expander, (((1,), (0,)), ((), ())),
            preferred_element_type=jnp.float32)
        o = jnp.sum(v2 * p_exp, axis=0)

        o_comm[0, i] = o
        s_comm[0, 0, i] = m
        s_comm[0, 1, i] = l

        @pl.when(i == nb - 1)
        def _comm():
            my_x = lax.axis_index("x")
            my_y = lax.axis_index("y")
            my_z = lax.axis_index("z")

            barrier = pltpu.get_barrier_semaphore()
            for off in (1, 2, 3):
                dst_z = lax.rem(my_z + off, N_Z)
                pl.semaphore_signal(
                    barrier, inc=1, device_id=(my_x, my_y, dst_z),
                    device_id_type=pl.DeviceIdType.MESH)
            pl.semaphore_wait(barrier, N_Z - 1)

            sends = []
            for off in (1, 2, 3):
                dst_z = lax.rem(my_z + off, N_Z)
                slot = N_Z - off
                for comm, ssem, rsem in ((o_comm, o_send, o_recv),
                                         (s_comm, s_send, s_recv)):
                    rd = pltpu.make_async_remote_copy(
                        src_ref=comm.at[0],
                        dst_ref=comm.at[slot],
                        send_sem=ssem.at[off - 1],
                        recv_sem=rsem.at[slot - 1],
                        device_id=(my_x, my_y, dst_z),
                        device_id_type=pl.DeviceIdType.MESH)
                    rd.start()
                    sends.append(rd)

            for slot in (1, 2, 3):
                for comm, ssem, rsem in ((o_comm, o_send, o_recv),
                                         (s_comm, s_send, s_recv)):
                    rd = pltpu.make_async_remote_copy(
                        src_ref=comm.at[0],
                        dst_ref=comm.at[slot],
                        send_sem=ssem.at[0],
                        recv_sem=rsem.at[slot - 1],
                        device_id=(my_x, my_y, my_z),
                        device_id_type=pl.DeviceIdType.MESH)
                    rd.wait_recv()

            for rd in sends:
                rd.wait_send()

            m_all = s_comm[:, 0]
            l_all = s_comm[:, 1]
            mx = jnp.max(m_all, axis=0)
            alpha = jnp.exp(m_all - mx[None])
            l_tot = jnp.sum(l_all * alpha, axis=0)
            alpha_exp = jnp.reshape(
                lax.dot_general(
                    jnp.reshape(alpha, (N_Z * b, h)), expander,
                    (((1,), (0,)), ((), ())),
                    preferred_element_type=jnp.float32),
                (N_Z, b, hd))
            l_exp = lax.dot_general(
                l_tot, expander, (((1,), (0,)), ((), ())),
                preferred_element_type=jnp.float32)
            o_all = o_comm[:, :, :]
            o_fin = jnp.sum(o_all * alpha_exp, axis=0) / l_exp
            out_ref[:, 0, :, :] = jnp.reshape(o_fin, (b, h, d))

    return pl.pallas_call(
        body,
        grid=(b,),
        in_specs=[
            pl.BlockSpec((b, qlen, h, d), lambda i: (0, 0, 0, 0)),
            pl.BlockSpec((1, k_per, h, d), lambda i: (i, 0, 0, 0)),
            pl.BlockSpec((1, k_per, h, d), lambda i: (i, 0, 0, 0)),
        ],
        out_specs=pl.BlockSpec((b, qlen, h, d), lambda i: (0, 0, 0, 0)),
        out_shape=jax.ShapeDtypeStruct((b, qlen, h, d), jnp.float32),
        scratch_shapes=[
            pltpu.VMEM((N_Z, b, h * d), jnp.float32),
            pltpu.VMEM((N_Z, 2, b, h), jnp.float32),
            pltpu.SemaphoreType.DMA((N_Z - 1,)),
            pltpu.SemaphoreType.DMA((N_Z - 1,)),
            pltpu.SemaphoreType.DMA((N_Z - 1,)),
            pltpu.SemaphoreType.DMA((N_Z - 1,)),
        ],
        compiler_params=pltpu.CompilerParams(
            collective_id=0,
            dimension_semantics=("arbitrary",),
            vmem_limit_bytes=64 * 1024 * 1024,
        ),
    )(Q, K, V)
